# baseline (speedup 1.0000x reference)
import jax
import jax.numpy as jnp
from jax import lax
from jax.experimental import pallas as pl
from jax.experimental.pallas import tpu as pltpu
from jax.experimental.pallas import tpu_sc as plsc

B = 16384
D = 64
V = 1000000
NC, NS, L = 2, 16, 16
NW = NC * NS
BPW = B // NW
NG = BPW // L
GW = D * L
CH = 128


def _sc_body(cust_hbm, arti_hbm, artj_hbm, wcf_hbm, waf_hbm,
             out_i_hbm, out_j_hbm,
             idx_c, idx_i, idx_j, off_c, off_i, off_j,
             cbuf, aibuf, ajbuf, dots_i, dots_j, sem_c, sem_i, sem_j):
    wid = lax.axis_index("s") * NC + lax.axis_index("c")
    base = wid * BPW

    pltpu.sync_copy(cust_hbm.at[pl.ds(base, BPW)], idx_c)
    pltpu.sync_copy(arti_hbm.at[pl.ds(base, BPW)], idx_i)
    pltpu.sync_copy(artj_hbm.at[pl.ds(base, BPW)], idx_j)

    def group(g, _):
        g16 = g * L
        vc = idx_c[pl.ds(g16, L)]
        vi = idx_i[pl.ds(g16, L)]
        vj = idx_j[pl.ds(g16, L)]
        for d in range(D):
            sl = pl.ds(d * L, L)
            dv = jnp.int32(d * V)
            off_c[sl] = vc + dv
            off_i[sl] = vi + dv
            off_j[sl] = vj + dv
        copies = []
        for k in range(GW // CH):
            sl = pl.ds(k * CH, CH)
            copies.append(pltpu.async_copy(
                wcf_hbm.at[off_c.at[sl]], cbuf.at[sl], sem_c))
            copies.append(pltpu.async_copy(
                waf_hbm.at[off_i.at[sl]], aibuf.at[sl], sem_i))
            copies.append(pltpu.async_copy(
                waf_hbm.at[off_j.at[sl]], ajbuf.at[sl], sem_j))
        for c in copies:
            c.wait()
        cv = cbuf[pl.ds(0, L)]
        acc_i = cv * aibuf[pl.ds(0, L)]
        acc_j = cv * ajbuf[pl.ds(0, L)]
        for d in range(1, D):
            sl = pl.ds(d * L, L)
            cv = cbuf[sl]
            acc_i = acc_i + cv * aibuf[sl]
            acc_j = acc_j + cv * ajbuf[sl]
        out_sl = pl.ds(g16, L)
        dots_i[out_sl] = 1.0 / (1.0 + jnp.exp(-acc_i))
        dots_j[out_sl] = 1.0 / (1.0 + jnp.exp(-acc_j))
        return 0

    lax.fori_loop(0, NG, group, 0)

    out = pl.ds(base, BPW)
    pltpu.sync_copy(dots_i, out_i_hbm.at[out])
    pltpu.sync_copy(dots_j, out_j_hbm.at[out])


@jax.jit
def _sc_call(customer, article_i, article_j, wcf, waf):
    mesh = plsc.VectorSubcoreMesh(core_axis_name="c", subcore_axis_name="s")
    f = pl.kernel(
        _sc_body,
        out_type=(
            jax.ShapeDtypeStruct((B,), jnp.float32),
            jax.ShapeDtypeStruct((B,), jnp.float32),
        ),
        mesh=mesh,
        compiler_params=pltpu.CompilerParams(
            needs_layout_passes=False, use_tc_tiling_on_sc=False),
        scratch_types=[
            pltpu.VMEM((BPW,), jnp.int32),
            pltpu.VMEM((BPW,), jnp.int32),
            pltpu.VMEM((BPW,), jnp.int32),
            pltpu.VMEM((GW,), jnp.int32),
            pltpu.VMEM((GW,), jnp.int32),
            pltpu.VMEM((GW,), jnp.int32),
            pltpu.VMEM((GW,), jnp.float32),
            pltpu.VMEM((GW,), jnp.float32),
            pltpu.VMEM((GW,), jnp.float32),
            pltpu.VMEM((BPW,), jnp.float32),
            pltpu.VMEM((BPW,), jnp.float32),
            pltpu.SemaphoreType.DMA,
            pltpu.SemaphoreType.DMA,
            pltpu.SemaphoreType.DMA,
        ],
    )
    return f(customer, article_i, article_j, wcf, waf)


def kernel(customer, article_i, article_j, W_customer, W_article):
    return _sc_call(customer, article_i, article_j,
                    W_customer.T.reshape(-1), W_article.T.reshape(-1))


# speedup vs baseline: 9.1212x; 9.1212x over previous
import jax
import jax.numpy as jnp
from jax import lax
from jax.experimental import pallas as pl
from jax.experimental.pallas import tpu as pltpu
from jax.experimental.pallas import tpu_sc as plsc

B = 16384
D = 64
NC, NS, L = 2, 16, 16
NW = NC * NS
BPW = B // NW
CHUNK = 128
NCHUNK = BPW // CHUNK
TP = L + 1


def _sc_body(cust_hbm, arti_hbm, artj_hbm, wc_hbm, wa_hbm,
             out_i_hbm, out_j_hbm,
             idx_c, idx_i, idx_j, c_rows, ai_rows, aj_rows,
             dots_i, dots_j, ti, tj, sem0, sem1, sem2, sem3):
    wid = lax.axis_index("s") * NC + lax.axis_index("c")
    row4 = wid * NCHUNK

    pltpu.sync_copy(cust_hbm.at[pl.ds(row4, NCHUNK)], idx_c)
    pltpu.sync_copy(arti_hbm.at[pl.ds(row4, NCHUNK)], idx_i)
    pltpu.sync_copy(artj_hbm.at[pl.ds(row4, NCHUNK)], idx_j)

    sems = (sem0, sem1, sem2, sem3)
    copies = []
    for k in range(NCHUNK):
        dst = pl.ds(k * CHUNK, CHUNK)
        copies.append((
            pltpu.async_copy(wc_hbm.at[idx_c.at[k]], c_rows.at[dst], sems[k]),
            pltpu.async_copy(wa_hbm.at[idx_i.at[k]], ai_rows.at[dst], sems[k]),
            pltpu.async_copy(wa_hbm.at[idx_j.at[k]], aj_rows.at[dst], sems[k]),
        ))

    scat_base = lax.iota(jnp.int32, L) * TP

    def block_body(blk, _):
        r0 = blk * L
        for r_local in range(L):
            r = r0 + r_local
            sl0 = pl.ds(0 * L, L)
            cv = c_rows[r, sl0]
            s_i = cv * ai_rows[r, sl0]
            s_j = cv * aj_rows[r, sl0]
            for seg in range(1, D // L):
                sl = pl.ds(seg * L, L)
                cv = c_rows[r, sl]
                s_i = s_i + cv * ai_rows[r, sl]
                s_j = s_j + cv * aj_rows[r, sl]
            idx = scat_base + r_local
            plsc.store_scatter(ti, [idx], s_i)
            plsc.store_scatter(tj, [idx], s_j)
        acc_i = ti[pl.ds(0, L)]
        acc_j = tj[pl.ds(0, L)]
        for l in range(1, L):
            acc_i = acc_i + ti[pl.ds(l * TP, L)]
            acc_j = acc_j + tj[pl.ds(l * TP, L)]
        out_sl = pl.ds(r0, L)
        dots_i[out_sl] = 1.0 / (1.0 + jnp.exp(-acc_i))
        dots_j[out_sl] = 1.0 / (1.0 + jnp.exp(-acc_j))
        return 0

    blocks_per_chunk = CHUNK // L
    for k in range(NCHUNK):
        for c in copies[k]:
            c.wait()
        lax.fori_loop(k * blocks_per_chunk, (k + 1) * blocks_per_chunk,
                      block_body, 0)

    out = pl.ds(wid * BPW, BPW)
    pltpu.sync_copy(dots_i, out_i_hbm.at[out])
    pltpu.sync_copy(dots_j, out_j_hbm.at[out])


@jax.jit
def _sc_call(cust2d, arti2d, artj2d, wc, wa):
    mesh = plsc.VectorSubcoreMesh(core_axis_name="c", subcore_axis_name="s")
    f = pl.kernel(
        _sc_body,
        out_type=(
            jax.ShapeDtypeStruct((B,), jnp.float32),
            jax.ShapeDtypeStruct((B,), jnp.float32),
        ),
        mesh=mesh,
        compiler_params=pltpu.CompilerParams(
            needs_layout_passes=False, use_tc_tiling_on_sc=False),
        scratch_types=[
            pltpu.VMEM((NCHUNK, CHUNK), jnp.int32),
            pltpu.VMEM((NCHUNK, CHUNK), jnp.int32),
            pltpu.VMEM((NCHUNK, CHUNK), jnp.int32),
            pltpu.VMEM((BPW, D), jnp.float32),
            pltpu.VMEM((BPW, D), jnp.float32),
            pltpu.VMEM((BPW, D), jnp.float32),
            pltpu.VMEM((BPW,), jnp.float32),
            pltpu.VMEM((BPW,), jnp.float32),
            pltpu.VMEM((L * TP,), jnp.float32),
            pltpu.VMEM((L * TP,), jnp.float32),
            pltpu.SemaphoreType.DMA,
            pltpu.SemaphoreType.DMA,
            pltpu.SemaphoreType.DMA,
            pltpu.SemaphoreType.DMA,
        ],
    )
    return f(cust2d, arti2d, artj2d, wc, wa)


def kernel(customer, article_i, article_j, W_customer, W_article):
    cust2d = customer.reshape(B // CHUNK, CHUNK)
    arti2d = article_i.reshape(B // CHUNK, CHUNK)
    artj2d = article_j.reshape(B // CHUNK, CHUNK)
    return _sc_call(cust2d, arti2d, artj2d, W_customer, W_article)


# speedup vs baseline: 11.0237x; 1.2086x over previous
import jax
import jax.numpy as jnp
from jax import lax
from jax.experimental import pallas as pl
from jax.experimental.pallas import tpu as pltpu
from jax.experimental.pallas import tpu_sc as plsc

B = 16384
D = 64
W = 2 * D
V = 1000000
NC, NS, L = 2, 16, 16
NW = NC * NS
BPW = B // NW
CHUNK = 128
NCHUNK = BPW // CHUNK
TP = L + 1


def _sc_body(cust_hbm, arti_hbm, artj_hbm, wb_hbm,
             out_i_hbm, out_j_hbm,
             idx_c, idx_i, idx_j,
             c0, c1, a0, a1, b0, b1,
             dots_i, dots_j, ti, tj, sem0, sem1, sem2, sem3):
    wid = lax.axis_index("s") * NC + lax.axis_index("c")
    base = wid * BPW
    row4 = wid * NCHUNK

    pltpu.sync_copy(cust_hbm.at[pl.ds(row4, NCHUNK)], idx_c)
    pltpu.sync_copy(arti_hbm.at[pl.ds(row4, NCHUNK)], idx_i)
    pltpu.sync_copy(artj_hbm.at[pl.ds(row4, NCHUNK)], idx_j)

    cbufs, abufs, bbufs = (c0, c1), (a0, a1), (b0, b1)
    sems = (sem0, sem1, sem2, sem3)

    def fire(k):
        return (
            pltpu.async_copy(wb_hbm.at[idx_c.at[k]], cbufs[k % 2], sems[k]),
            pltpu.async_copy(wb_hbm.at[idx_i.at[k]], abufs[k % 2], sems[k]),
            pltpu.async_copy(wb_hbm.at[idx_j.at[k]], bbufs[k % 2], sems[k]),
        )

    scat_base = lax.iota(jnp.int32, L) * TP
    inflight = {0: fire(0), 1: fire(1)}

    for k in range(NCHUNK):
        for c in inflight.pop(k):
            c.wait()
        c_rows, ai_rows, aj_rows = cbufs[k % 2], abufs[k % 2], bbufs[k % 2]

        def block_body(blk, _):
            r0 = blk * L
            for r_local in range(L):
                r = r0 + r_local
                sl0 = pl.ds(0, L)
                sa0 = pl.ds(D, L)
                cv = c_rows[r, sl0]
                s_i = cv * ai_rows[r, sa0]
                s_j = cv * aj_rows[r, sa0]
                for seg in range(1, D // L):
                    slc = pl.ds(seg * L, L)
                    sla = pl.ds(D + seg * L, L)
                    cv = c_rows[r, slc]
                    s_i = s_i + cv * ai_rows[r, sla]
                    s_j = s_j + cv * aj_rows[r, sla]
                idx = scat_base + r_local
                plsc.store_scatter(ti, [idx], s_i)
                plsc.store_scatter(tj, [idx], s_j)
            acc_i = ti[pl.ds(0, L)]
            acc_j = tj[pl.ds(0, L)]
            for l in range(1, L):
                acc_i = acc_i + ti[pl.ds(l * TP, L)]
                acc_j = acc_j + tj[pl.ds(l * TP, L)]
            out_sl = pl.ds(k * CHUNK + r0, L)
            dots_i[out_sl] = 1.0 / (1.0 + jnp.exp(-acc_i))
            dots_j[out_sl] = 1.0 / (1.0 + jnp.exp(-acc_j))
            return 0

        lax.fori_loop(0, CHUNK // L, block_body, 0)
        if k + 2 < NCHUNK:
            inflight[k + 2] = fire(k + 2)

    out = pl.ds(base, BPW)
    pltpu.sync_copy(dots_i, out_i_hbm.at[out])
    pltpu.sync_copy(dots_j, out_j_hbm.at[out])


@jax.jit
def _sc_call(customer, article_i, article_j, wboth):
    mesh = plsc.VectorSubcoreMesh(core_axis_name="c", subcore_axis_name="s")
    f = pl.kernel(
        _sc_body,
        out_type=(
            jax.ShapeDtypeStruct((B,), jnp.float32),
            jax.ShapeDtypeStruct((B,), jnp.float32),
        ),
        mesh=mesh,
        compiler_params=pltpu.CompilerParams(needs_layout_passes=False),
        scratch_types=[
            pltpu.VMEM((NCHUNK, CHUNK), jnp.int32),
            pltpu.VMEM((NCHUNK, CHUNK), jnp.int32),
            pltpu.VMEM((NCHUNK, CHUNK), jnp.int32),
            pltpu.VMEM((CHUNK, W), jnp.float32),
            pltpu.VMEM((CHUNK, W), jnp.float32),
            pltpu.VMEM((CHUNK, W), jnp.float32),
            pltpu.VMEM((CHUNK, W), jnp.float32),
            pltpu.VMEM((CHUNK, W), jnp.float32),
            pltpu.VMEM((CHUNK, W), jnp.float32),
            pltpu.VMEM((BPW,), jnp.float32),
            pltpu.VMEM((BPW,), jnp.float32),
            pltpu.VMEM((L * TP,), jnp.float32),
            pltpu.VMEM((L * TP,), jnp.float32),
            pltpu.SemaphoreType.DMA,
            pltpu.SemaphoreType.DMA,
            pltpu.SemaphoreType.DMA,
            pltpu.SemaphoreType.DMA,
        ],
    )
    return f(customer, article_i, article_j, wboth)


def kernel(customer, article_i, article_j, W_customer, W_article):
    cust2d = customer.reshape(B // CHUNK, CHUNK)
    arti2d = article_i.reshape(B // CHUNK, CHUNK)
    artj2d = article_j.reshape(B // CHUNK, CHUNK)
    wboth = (jnp.pad(W_customer, ((0, 0), (0, D)))
             + jnp.pad(W_article, ((0, 0), (D, 0))))
    return _sc_call(cust2d, arti2d, artj2d, wboth)
